# Initial kernel scaffold; baseline (speedup 1.0000x reference)
import jax
import jax.numpy as jnp
from jax.experimental import pallas as pl


def kernel(x, fb_filters, rw1, rb1, rw2, rb2, rw3, rb3, conv_w, conv_b, d0w, d0b, c0w, c0b, d1w, d1b, c1w, c1b, d2w, d2b, c2w, c2b, up_w, up_b, tv_w, tv_b, atoms):
    raise NotImplementedError("write your pallas kernel here")


# speedup vs baseline: 3.2600x; 3.2600x over previous
import jax
import jax.numpy as jnp
from jax import lax
from jax.experimental import pallas as pl

_NS = 32768
_K = 512
_NB = 128
_F = 128
_STEP = 256
_NA = 2048
_AS = 2048
_TOPK = 32


def _leaky(x):
    return jnp.where(x >= 0, x, 0.2 * x)


def _dotg(a, b):
    return lax.dot_general(a, b, (((1,), (1,)), ((), ())),
                           preferred_element_type=jnp.float32)


def _k1_body(x_ref, w_ref, o_ref):
    o_ref[...] = jnp.dot(x_ref[...], w_ref[...],
                         preferred_element_type=jnp.float32)


def _k2_body(sa_ref, sb_ref, dre_ref, dim_ref, w1_ref, b1_ref, w2_ref,
             b2_ref, w3_ref, b3_ref, o_ref):
    fa = sa_ref[0].reshape(256, 128)
    fb = sb_ref[0].reshape(256, 128)
    frame = jnp.concatenate([fa, fb], axis=0)
    re = jnp.dot(dre_ref[...], frame, preferred_element_type=jnp.float32)
    im = jnp.dot(dim_ref[...], frame, preferred_element_type=jnp.float32)
    mag = jnp.sqrt(re * re + im * im)
    h = lax.dot_general(mag, w1_ref[...], (((0,), (0,)), ((), ())),
                        preferred_element_type=jnp.float32) + b1_ref[...]
    h = _leaky(h)
    h = jnp.dot(h, w2_ref[...], preferred_element_type=jnp.float32) + b2_ref[...]
    h = _leaky(h)
    h = jnp.dot(h, w3_ref[...], preferred_element_type=jnp.float32) + b3_ref[...]
    o_ref[0, 0] = h


def _shift_rows(h, off, n):
    if off == 0:
        return h
    z = jnp.zeros((abs(off), h.shape[1]), h.dtype)
    if off < 0:
        return jnp.concatenate([z, h[: n + off, :]], axis=0)
    return jnp.concatenate([h[off:, :], z], axis=0)


def _k3_body(x_ref, cw1_ref, cb1_ref, dw_ref, db_ref, cw_ref, cb_ref,
             uw_ref, ub_ref, tw_ref, tb_ref, xf_ref, v_ref):
    n = 128
    h = _dotg(x_ref[0], cw1_ref[...]) + cb1_ref[...]
    for blk, d in enumerate((1, 3, 9)):
        orig = h
        left = _shift_rows(h, -d, n)
        right = _shift_rows(h, d, n)
        hh = (_dotg(left, dw_ref[blk, 0]) + _dotg(h, dw_ref[blk, 1])
              + _dotg(right, dw_ref[blk, 2]) + db_ref[blk])
        hh = _dotg(hh, cw_ref[blk]) + cb_ref[blk]
        h = _leaky(hh + orig)
    acc = jnp.zeros((n, _NA), jnp.float32) + ub_ref[...]
    for j in range(7):
        sh = _shift_rows(h, j - 3, n)
        acc = acc + _dotg(sh, uw_ref[j])
    xf_ref[0] = acc
    v_ref[0] = jnp.abs(_dotg(acc, tw_ref[...]) + tb_ref[...])


def _k4_body(xf_ref, v_ref, at_ref, o_ref):
    xf = xf_ref[0]
    v = v_ref[0]
    m = jnp.max(xf)
    s = jnp.sum(jnp.exp(xf - m))
    ii = (lax.broadcasted_iota(jnp.int32, (128, _NA), 0) * _NA
          + lax.broadcasted_iota(jnp.int32, (128, _NA), 1))
    iot = lax.broadcasted_iota(jnp.int32, (_TOPK, 1), 0)

    def body(i, carry):
        work, idxv, vvv = carry
        val = jnp.max(work)
        idx = jnp.min(jnp.where(work == val, ii, jnp.int32(2147483647)))
        vsel = jnp.sum(jnp.where(ii == idx, v, 0.0))
        vv = jnp.exp(val - m) / s * vsel
        idxv = jnp.where(iot == i, idx, idxv)
        vvv = jnp.where(iot == i, vv, vvv)
        work = jnp.where(ii == idx, jnp.float32(-3e38), work)
        return work, idxv, vvv

    _, idxv, vvv = lax.fori_loop(
        0, _TOPK, body,
        (xf, jnp.zeros((_TOPK, 1), jnp.int32), jnp.zeros((_TOPK, 1), jnp.float32)))

    pv = idxv // _NA
    av = idxv - pv * _NA
    a1 = (lax.broadcasted_iota(jnp.int32, (_TOPK, _NA), 1) == av
          ).astype(jnp.float32)
    sel = jnp.dot(a1, at_ref[...], preferred_element_type=jnp.float32)
    nr = jnp.sqrt(jnp.sum(sel * sel, axis=1, keepdims=True))
    smat = sel * (vvv / (nr + 1e-8))
    out = jnp.zeros((136, 256), jnp.float32)
    pio = lax.broadcasted_iota(jnp.int32, (_TOPK, 136), 1)
    for dr in range(8):
        pdr = (pio == pv + dr).astype(jnp.float32)
        out = out + lax.dot_general(
            pdr, smat[:, dr * 256:(dr + 1) * 256],
            (((0,), (0,)), ((), ())), preferred_element_type=jnp.float32)
    o_ref[0] = out


def kernel(x, fb_filters, rw1, rb1, rw2, rb2, rw3, rb3, conv_w, conv_b,
           d0w, d0b, c0w, c0b, d1w, d1b, c1w, c1b, d2w, d2b, c2w, c2b,
           up_w, up_b, tv_w, tv_b, atoms):
    B = x.shape[0]
    f32 = jnp.float32

    xpad = jnp.pad(x[:, 0, :], ((0, 0), (256, 256)))
    xb = xpad.reshape(B, 520, 64)
    x9 = jnp.concatenate([xb[:, i:i + 512, :] for i in range(9)],
                         axis=2).reshape(B * 512, 576)
    fbsq = fb_filters[:, 0, :]
    jr = jnp.arange(576)[:, None] - jnp.arange(64)[None, :]
    w9 = jnp.where(((jr >= 0) & (jr < _K))[:, :, None],
                   fbsq.T[jnp.clip(jr, 0, _K - 1)],
                   0.0).reshape(576, 64 * _NB)

    spec_cols = pl.pallas_call(
        _k1_body,
        grid=(8,),
        in_specs=[pl.BlockSpec((256, 576), lambda i: (i, 0)),
                  pl.BlockSpec((576, 8192), lambda i: (0, 0))],
        out_specs=pl.BlockSpec((256, 8192), lambda i: (i, 0)),
        out_shape=jax.ShapeDtypeStruct((B * 512, 8192), f32),
    )(x9, w9)
    spec_pad = jnp.pad(spec_cols.reshape(B, 512, 64, _NB),
                       ((0, 0), (0, 8), (0, 0), (0, 0)))

    t = jnp.arange(_K, dtype=f32)
    om = jnp.arange(264, dtype=f32)
    ang = 2.0 * jnp.pi * om[:, None] * t[None, :] / _K
    scale = 1.0 / jnp.sqrt(jnp.float32(_K))
    dre = jnp.cos(ang) * scale
    dim = -jnp.sin(ang) * scale
    rw1p = jnp.pad(rw1, ((0, 264 - 257), (0, 0)))

    h8 = pl.pallas_call(
        _k2_body,
        grid=(B, _F),
        in_specs=[
            pl.BlockSpec((1, 4, 64, _NB), lambda b, f: (b, f, 0, 0)),
            pl.BlockSpec((1, 4, 64, _NB), lambda b, f: (b, f + 1, 0, 0)),
            pl.BlockSpec((264, _K), lambda b, f: (0, 0)),
            pl.BlockSpec((264, _K), lambda b, f: (0, 0)),
            pl.BlockSpec((264, 128), lambda b, f: (0, 0)),
            pl.BlockSpec((1, 128), lambda b, f: (0, 0)),
            pl.BlockSpec((128, 128), lambda b, f: (0, 0)),
            pl.BlockSpec((1, 128), lambda b, f: (0, 0)),
            pl.BlockSpec((128, 8), lambda b, f: (0, 0)),
            pl.BlockSpec((1, 8), lambda b, f: (0, 0)),
        ],
        out_specs=pl.BlockSpec((1, 1, _NB, 8), lambda b, f: (b, f, 0, 0)),
        out_shape=jax.ShapeDtypeStruct((B, _F, _NB, 8), f32),
    )(spec_pad, spec_pad, dre, dim, rw1p, rb1[None, :], rw2, rb2[None, :],
      rw3, rb3[None, :])

    x3 = h8.transpose(0, 2, 3, 1).reshape(B, 128, 1024)

    dw = jnp.stack([d0w.transpose(2, 0, 1), d1w.transpose(2, 0, 1),
                    d2w.transpose(2, 0, 1)])
    db = jnp.stack([d0b, d1b, d2b])[:, None, :]
    cw = jnp.stack([c0w[:, :, 0], c1w[:, :, 0], c2w[:, :, 0]])
    cb = jnp.stack([c0b, c1b, c2b])[:, None, :]
    uw = up_w.transpose(2, 0, 1)

    xf, vmat = pl.pallas_call(
        _k3_body,
        grid=(B,),
        in_specs=[
            pl.BlockSpec((1, 128, 1024), lambda b: (b, 0, 0)),
            pl.BlockSpec((128, 1024), lambda b: (0, 0)),
            pl.BlockSpec((1, 128), lambda b: (0, 0)),
            pl.BlockSpec((3, 3, 128, 128), lambda b: (0, 0, 0, 0)),
            pl.BlockSpec((3, 1, 128), lambda b: (0, 0, 0)),
            pl.BlockSpec((3, 128, 128), lambda b: (0, 0, 0)),
            pl.BlockSpec((3, 1, 128), lambda b: (0, 0, 0)),
            pl.BlockSpec((7, _NA, 128), lambda b: (0, 0, 0)),
            pl.BlockSpec((1, _NA), lambda b: (0, 0)),
            pl.BlockSpec((_NA, _NA), lambda b: (0, 0)),
            pl.BlockSpec((1, _NA), lambda b: (0, 0)),
        ],
        out_specs=[pl.BlockSpec((1, 128, _NA), lambda b: (b, 0, 0)),
                   pl.BlockSpec((1, 128, _NA), lambda b: (b, 0, 0))],
        out_shape=[jax.ShapeDtypeStruct((B, 128, _NA), f32),
                   jax.ShapeDtypeStruct((B, 128, _NA), f32)],
    )(x3, conv_w[:, :, 0], conv_b[None, :], dw, db, cw, cb,
      uw, up_b[None, :], tv_w[:, :, 0], tv_b[None, :])

    out4 = pl.pallas_call(
        _k4_body,
        grid=(B,),
        in_specs=[
            pl.BlockSpec((1, 128, _NA), lambda b: (b, 0, 0)),
            pl.BlockSpec((1, 128, _NA), lambda b: (b, 0, 0)),
            pl.BlockSpec((_NA, _AS), lambda b: (0, 0)),
        ],
        out_specs=pl.BlockSpec((1, 136, 256), lambda b: (b, 0, 0)),
        out_shape=jax.ShapeDtypeStruct((B, 136, 256), f32),
    )(xf, vmat, atoms)

    return out4.reshape(B, 34816)[:, None, :_NS]


# speedup vs baseline: 5.1770x; 1.5880x over previous
import jax
import jax.numpy as jnp
from jax import lax
from jax.experimental import pallas as pl

_NS = 32768
_K = 512
_NB = 128
_F = 128
_STEP = 256
_NA = 2048
_AS = 2048
_TOPK = 32


def _leaky(x):
    return jnp.where(x >= 0, x, 0.2 * x)


def _dotg(a, b):
    return lax.dot_general(a, b, (((1,), (1,)), ((), ())),
                           preferred_element_type=jnp.float32)


def _k1_body(x_ref, w_ref, o_ref):
    o_ref[...] = jnp.dot(x_ref[...], w_ref[...],
                         preferred_element_type=jnp.float32)


def _k2_body(sa_ref, sb_ref, dre_ref, dim_ref, w1_ref, b1_ref, w2_ref,
             b2_ref, w3_ref, b3_ref, o_ref):
    ra = sa_ref[0].reshape(4096, 128)
    rb = sb_ref[0].reshape(4096, 128)
    cols = [ra[256 * l:256 * l + 512, :] for l in range(15)]
    cols.append(jnp.concatenate([ra[3840:4096, :], rb[0:256, :]], axis=0))
    fmat = jnp.concatenate(cols, axis=1)
    re = jnp.dot(dre_ref[...], fmat, preferred_element_type=jnp.float32)
    im = jnp.dot(dim_ref[...], fmat, preferred_element_type=jnp.float32)
    mag = jnp.sqrt(re * re + im * im)
    h = lax.dot_general(mag, w1_ref[...], (((0,), (0,)), ((), ())),
                        preferred_element_type=jnp.float32) + b1_ref[...]
    h = _leaky(h)
    h = jnp.dot(h, w2_ref[...], preferred_element_type=jnp.float32) + b2_ref[...]
    h = _leaky(h)
    h = jnp.dot(h, w3_ref[...], preferred_element_type=jnp.float32) + b3_ref[...]
    o_ref[0] = h.reshape(16, 128, 8)


def _shift_rows(h, off, n):
    if off == 0:
        return h
    z = jnp.zeros((abs(off), h.shape[1]), h.dtype)
    if off < 0:
        return jnp.concatenate([z, h[: n + off, :]], axis=0)
    return jnp.concatenate([h[off:, :], z], axis=0)


def _k3_body(x_ref, cw1_ref, cb1_ref, dw_ref, db_ref, cw_ref, cb_ref,
             uw_ref, ub_ref, tw_ref, tb_ref, xf_ref, v_ref):
    n = 128
    h = _dotg(x_ref[0], cw1_ref[...]) + cb1_ref[...]
    for blk, d in enumerate((1, 3, 9)):
        orig = h
        left = _shift_rows(h, -d, n)
        right = _shift_rows(h, d, n)
        hh = (_dotg(left, dw_ref[blk, 0]) + _dotg(h, dw_ref[blk, 1])
              + _dotg(right, dw_ref[blk, 2]) + db_ref[blk])
        hh = _dotg(hh, cw_ref[blk]) + cb_ref[blk]
        h = _leaky(hh + orig)
    acc = jnp.zeros((n, _NA), jnp.float32) + ub_ref[...]
    for j in range(7):
        sh = _shift_rows(h, j - 3, n)
        acc = acc + _dotg(sh, uw_ref[j])
    xf_ref[0] = acc
    v_ref[0] = jnp.abs(_dotg(acc, tw_ref[...]) + tb_ref[...])


def _k4_body(xf_ref, v_ref, at_ref, o_ref):
    xf = xf_ref[0]
    v = v_ref[0]
    m = jnp.max(xf)
    s = jnp.sum(jnp.exp(xf - m))
    ii = (lax.broadcasted_iota(jnp.int32, (128, _NA), 0) * _NA
          + lax.broadcasted_iota(jnp.int32, (128, _NA), 1))
    iot = lax.broadcasted_iota(jnp.int32, (_TOPK, 1), 0)

    def body(i, carry):
        work, idxv, vvv = carry
        val = jnp.max(work)
        idx = jnp.min(jnp.where(work == val, ii, jnp.int32(2147483647)))
        vsel = jnp.sum(jnp.where(ii == idx, v, 0.0))
        vv = jnp.exp(val - m) / s * vsel
        idxv = jnp.where(iot == i, idx, idxv)
        vvv = jnp.where(iot == i, vv, vvv)
        work = jnp.where(ii == idx, jnp.float32(-3e38), work)
        return work, idxv, vvv

    _, idxv, vvv = lax.fori_loop(
        0, _TOPK, body,
        (xf, jnp.zeros((_TOPK, 1), jnp.int32), jnp.zeros((_TOPK, 1), jnp.float32)))

    pv = idxv // _NA
    av = idxv - pv * _NA
    a1 = (lax.broadcasted_iota(jnp.int32, (_TOPK, _NA), 1) == av
          ).astype(jnp.float32)
    sel = jnp.dot(a1, at_ref[...], preferred_element_type=jnp.float32)
    nr = jnp.sqrt(jnp.sum(sel * sel, axis=1, keepdims=True))
    smat = sel * (vvv / (nr + 1e-8))
    out = jnp.zeros((136, 256), jnp.float32)
    pio = lax.broadcasted_iota(jnp.int32, (_TOPK, 136), 1)
    for dr in range(8):
        pdr = (pio == pv + dr).astype(jnp.float32)
        out = out + lax.dot_general(
            pdr, smat[:, dr * 256:(dr + 1) * 256],
            (((0,), (0,)), ((), ())), preferred_element_type=jnp.float32)
    o_ref[0] = out


def kernel(x, fb_filters, rw1, rb1, rw2, rb2, rw3, rb3, conv_w, conv_b,
           d0w, d0b, c0w, c0b, d1w, d1b, c1w, c1b, d2w, d2b, c2w, c2b,
           up_w, up_b, tv_w, tv_b, atoms):
    B = x.shape[0]
    f32 = jnp.float32

    xpad = jnp.pad(x[:, 0, :], ((0, 0), (256, 256)))
    xb = xpad.reshape(B, 520, 64)
    x9 = jnp.concatenate([xb[:, i:i + 512, :] for i in range(9)],
                         axis=2).reshape(B * 512, 576)
    fbsq = fb_filters[:, 0, :]
    jr = jnp.arange(576)[:, None] - jnp.arange(64)[None, :]
    w9 = jnp.where(((jr >= 0) & (jr < _K))[:, :, None],
                   fbsq.T[jnp.clip(jr, 0, _K - 1)],
                   0.0).reshape(576, 64 * _NB)

    spec_cols = pl.pallas_call(
        _k1_body,
        grid=(8,),
        in_specs=[pl.BlockSpec((256, 576), lambda i: (i, 0)),
                  pl.BlockSpec((576, 8192), lambda i: (0, 0))],
        out_specs=pl.BlockSpec((256, 8192), lambda i: (i, 0)),
        out_shape=jax.ShapeDtypeStruct((B * 512, 8192), f32),
    )(x9, w9)
    spec_pad = jnp.pad(spec_cols.reshape(B, 512, 64, _NB),
                       ((0, 0), (0, 64), (0, 0), (0, 0)))

    t = jnp.arange(_K, dtype=f32)
    om = jnp.arange(264, dtype=f32)
    ang = 2.0 * jnp.pi * om[:, None] * t[None, :] / _K
    scale = 1.0 / jnp.sqrt(jnp.float32(_K))
    dre = jnp.cos(ang) * scale
    dim = -jnp.sin(ang) * scale
    rw1p = jnp.pad(rw1, ((0, 264 - 257), (0, 0)))

    h8 = pl.pallas_call(
        _k2_body,
        grid=(B, 8),
        in_specs=[
            pl.BlockSpec((1, 64, 64, _NB), lambda b, g: (b, g, 0, 0)),
            pl.BlockSpec((1, 64, 64, _NB), lambda b, g: (b, g + 1, 0, 0)),
            pl.BlockSpec((264, _K), lambda b, g: (0, 0)),
            pl.BlockSpec((264, _K), lambda b, g: (0, 0)),
            pl.BlockSpec((264, 128), lambda b, g: (0, 0)),
            pl.BlockSpec((1, 128), lambda b, g: (0, 0)),
            pl.BlockSpec((128, 128), lambda b, g: (0, 0)),
            pl.BlockSpec((1, 128), lambda b, g: (0, 0)),
            pl.BlockSpec((128, 8), lambda b, g: (0, 0)),
            pl.BlockSpec((1, 8), lambda b, g: (0, 0)),
        ],
        out_specs=pl.BlockSpec((1, 16, _NB, 8), lambda b, g: (b, g, 0, 0)),
        out_shape=jax.ShapeDtypeStruct((B, _F, _NB, 8), f32),
    )(spec_pad, spec_pad, dre, dim, rw1p, rb1[None, :], rw2, rb2[None, :],
      rw3, rb3[None, :])

    x3 = h8.transpose(0, 2, 3, 1).reshape(B, 128, 1024)

    dw = jnp.stack([d0w.transpose(2, 0, 1), d1w.transpose(2, 0, 1),
                    d2w.transpose(2, 0, 1)])
    db = jnp.stack([d0b, d1b, d2b])[:, None, :]
    cw = jnp.stack([c0w[:, :, 0], c1w[:, :, 0], c2w[:, :, 0]])
    cb = jnp.stack([c0b, c1b, c2b])[:, None, :]
    uw = up_w.transpose(2, 0, 1)

    xf, vmat = pl.pallas_call(
        _k3_body,
        grid=(B,),
        in_specs=[
            pl.BlockSpec((1, 128, 1024), lambda b: (b, 0, 0)),
            pl.BlockSpec((128, 1024), lambda b: (0, 0)),
            pl.BlockSpec((1, 128), lambda b: (0, 0)),
            pl.BlockSpec((3, 3, 128, 128), lambda b: (0, 0, 0, 0)),
            pl.BlockSpec((3, 1, 128), lambda b: (0, 0, 0)),
            pl.BlockSpec((3, 128, 128), lambda b: (0, 0, 0)),
            pl.BlockSpec((3, 1, 128), lambda b: (0, 0, 0)),
            pl.BlockSpec((7, _NA, 128), lambda b: (0, 0, 0)),
            pl.BlockSpec((1, _NA), lambda b: (0, 0)),
            pl.BlockSpec((_NA, _NA), lambda b: (0, 0)),
            pl.BlockSpec((1, _NA), lambda b: (0, 0)),
        ],
        out_specs=[pl.BlockSpec((1, 128, _NA), lambda b: (b, 0, 0)),
                   pl.BlockSpec((1, 128, _NA), lambda b: (b, 0, 0))],
        out_shape=[jax.ShapeDtypeStruct((B, 128, _NA), f32),
                   jax.ShapeDtypeStruct((B, 128, _NA), f32)],
    )(x3, conv_w[:, :, 0], conv_b[None, :], dw, db, cw, cb,
      uw, up_b[None, :], tv_w[:, :, 0], tv_b[None, :])

    out4 = pl.pallas_call(
        _k4_body,
        grid=(B,),
        in_specs=[
            pl.BlockSpec((1, 128, _NA), lambda b: (b, 0, 0)),
            pl.BlockSpec((1, 128, _NA), lambda b: (b, 0, 0)),
            pl.BlockSpec((_NA, _AS), lambda b: (0, 0)),
        ],
        out_specs=pl.BlockSpec((1, 136, 256), lambda b: (b, 0, 0)),
        out_shape=jax.ShapeDtypeStruct((B, 136, 256), f32),
    )(xf, vmat, atoms)

    return out4.reshape(B, 34816)[:, None, :_NS]
